# baseline (speedup 1.0000x reference)
import functools

import jax
import jax.numpy as jnp
from jax.experimental import pallas as pl

N = 2048
K = 16
AZ_BINS = 8
EL_BINS = 2
RAD_BINS = 2
HIST_BINS = 11
TOTAL_BINS = AZ_BINS * EL_BINS * RAD_BINS * HIST_BINS
ROWS = 256
NB = N // ROWS
BIG = 3.0e38


def _eigh3x3(a00, a01, a02, a11, a12, a22, sweeps=5):
    w = {(0, 0): a00, (0, 1): a01, (0, 2): a02,
         (1, 1): a11, (1, 2): a12, (2, 2): a22}
    one = jnp.ones_like(a00)
    zero = jnp.zeros_like(a00)
    v = {}
    for i in range(3):
        for j in range(3):
            v[(i, j)] = one if i == j else zero

    def W(i, j):
        return w[(i, j)] if i <= j else w[(j, i)]

    for _ in range(sweeps):
        for (p, q) in ((0, 2), (2, 1), (0, 1)):
            wpp, wqq, wpq = W(p, p), W(q, q), W(p, q)
            tau = (wqq - wpp) / (2.0 * wpq)
            t = jnp.sign(tau) / (jnp.abs(tau) + jnp.sqrt(1.0 + tau * tau))
            t = jnp.where(wpq != 0.0, t, 0.0)
            c = 1.0 / jnp.sqrt(1.0 + t * t)
            s = t * c
            r = 3 - p - q
            wpr, wqr = W(p, r), W(q, r)
            b_pp = c * wpp - s * wpq
            b_pq = c * wpq - s * wqq
            b_qp = s * wpp + c * wpq
            b_qq = s * wpq + c * wqq
            nw = dict(w)
            nw[(p, p)] = c * b_pp - s * b_pq
            nw[(q, q)] = s * b_qp + c * b_qq
            nw[(min(p, q), max(p, q))] = (s * b_pp + c * b_pq) if p < q \
                else (c * b_qp - s * b_qq)
            nw[(min(p, r), max(p, r))] = c * wpr - s * wqr
            nw[(min(q, r), max(q, r))] = s * wpr + c * wqr
            w = {k: nw[k] for k in w}
            nv = dict(v)
            for i in range(3):
                nv[(i, p)] = c * v[(i, p)] - s * v[(i, q)]
                nv[(i, q)] = s * v[(i, p)] + c * v[(i, q)]
            v = nv

    e = [W(0, 0), W(1, 1), W(2, 2)]
    emin = jnp.minimum(jnp.minimum(e[0], e[1]), e[2])
    emax = jnp.maximum(jnp.maximum(e[0], e[1]), e[2])
    zcol = [jnp.where(e[0] == emin, v[(i, 0)],
                      jnp.where(e[1] == emin, v[(i, 1)], v[(i, 2)]))
            for i in range(3)]
    xcol = [jnp.where(e[2] == emax, v[(i, 2)],
                      jnp.where(e[1] == emax, v[(i, 1)], v[(i, 0)]))
            for i in range(3)]
    return zcol, xcol


def _onehot_gather(onehot_f, table):
    return jax.lax.dot_general(
        onehot_f, table, (((1,), (0,)), ((), ())),
        preferred_element_type=jnp.float32,
        precision=jax.lax.Precision.HIGHEST)


def _lrf_kernel(vrow_ref, vfull_ref, vt_ref, lz_ref, ly_ref, lx_ref, nrm_ref,
                idx_ref, rmax_ref):
    vb = vrow_ref[0]
    xb = vb[:, 0:1]
    yb = vb[:, 1:2]
    zb = vb[:, 2:3]
    xa = vt_ref[0, 0:1, :]
    ya = vt_ref[0, 1:2, :]
    za = vt_ref[0, 2:3, :]

    d2 = (xb - xa) ** 2 + (yb - ya) ** 2 + (zb - za) ** 2

    row0 = pl.program_id(1) * ROWS
    rows = jax.lax.broadcasted_iota(jnp.int32, (ROWS, 1), 0) + row0
    cols = jax.lax.broadcasted_iota(jnp.int32, (1, N), 1)
    vals = jnp.where(cols == rows, BIG, d2)

    vfull = vfull_ref[0]
    dxs, dys, dzs, idxs = [], [], [], []
    for _ in range(K):
        m = jnp.min(vals, axis=1, keepdims=True)
        cand = jnp.where(vals == m, cols, N * 2)
        j = jnp.min(cand, axis=1, keepdims=True)
        onehot = cols == j
        nxyz = _onehot_gather(onehot.astype(jnp.float32), vfull)
        vals = jnp.where(onehot, BIG, vals)
        dxs.append(nxyz[:, 0:1] - xb)
        dys.append(nxyz[:, 1:2] - yb)
        dzs.append(nxyz[:, 2:3] - zb)
        idxs.append(j)

    dx = jnp.concatenate(dxs, axis=1)
    dy = jnp.concatenate(dys, axis=1)
    dz = jnp.concatenate(dzs, axis=1)
    idx = jnp.concatenate(idxs, axis=1)

    dist = jnp.sqrt(dx * dx + dy * dy + dz * dz + 1e-12)
    rloc = jnp.max(dist, axis=1, keepdims=True)
    w = jnp.maximum(rloc - dist, 0.0)
    sw = jnp.sum(w, axis=1, keepdims=True) + 1e-12

    def rne(t):
        return t.astype(jnp.bfloat16).astype(jnp.float32)

    wdx = rne(w * dx)
    wdy = rne(w * dy)
    wdz = rne(w * dz)
    rdx = rne(dx)
    rdy = rne(dy)
    rdz = rne(dz)

    def ksum(t):
        return jnp.sum(t, axis=1, keepdims=True)

    cxx = ksum(wdx * rdx) / sw + 1e-8
    cyy = ksum(wdy * rdy) / sw + 1e-8
    czz = ksum(wdz * rdz) / sw + 1e-8
    cxy = (ksum(wdx * rdy) / sw + ksum(wdy * rdx) / sw) / 2.0
    cxz = (ksum(wdx * rdz) / sw + ksum(wdz * rdx) / sw) / 2.0
    cyz = (ksum(wdy * rdz) / sw + ksum(wdz * rdy) / sw) / 2.0

    zc, xc = _eigh3x3(cxx, cxy, cxz, cyy, cyz, czz)

    dotx = rdx * rne(xc[0]) + rdy * rne(xc[1]) + rdz * rne(xc[2])
    sx = jnp.sum((dotx >= 0.0).astype(jnp.int32), axis=1, keepdims=True)
    fx = jnp.where(2 * sx >= K, 1.0, -1.0)
    xc = [x * fx for x in xc]
    dotz = rdx * rne(zc[0]) + rdy * rne(zc[1]) + rdz * rne(zc[2])
    sz = jnp.sum((dotz >= 0.0).astype(jnp.int32), axis=1, keepdims=True)
    fz = jnp.where(2 * sz >= K, 1.0, -1.0)
    zc = [z * fz for z in zc]

    yc = [zc[1] * xc[2] - zc[2] * xc[1],
          zc[2] * xc[0] - zc[0] * xc[2],
          zc[0] * xc[1] - zc[1] * xc[0]]

    lz = rdx * rne(zc[0]) + rdy * rne(zc[1]) + rdz * rne(zc[2])
    ly = rdx * rne(yc[0]) + rdy * rne(yc[1]) + rdz * rne(yc[2])
    lx = rdx * rne(xc[0]) + rdy * rne(xc[1]) + rdz * rne(xc[2])

    lz_ref[0] = lz
    ly_ref[0] = ly
    lx_ref[0] = lx
    nrm_ref[0] = jnp.concatenate(zc, axis=1)
    idx_ref[0] = idx
    rmax_ref[0] = jnp.max(jnp.maximum(jnp.maximum(lz, ly), lx), axis=1,
                          keepdims=True)


def _descr_kernel(nrow_ref, nfull_ref, idx_ref, lz_ref, ly_ref, lx_ref,
                  rmax_ref, out_ref):
    radius = jnp.max(rmax_ref[0], axis=1, keepdims=True)
    radeps = radius + 1e-12

    nb = nrow_ref[0]
    zx = nb[:, 0:1]
    zy = nb[:, 1:2]
    zz = nb[:, 2:3]
    nfull = nfull_ref[0]
    idx = idx_ref[0]
    cols = jax.lax.broadcasted_iota(jnp.int32, (1, N), 1)

    hists = []
    for k in range(K):
        jk = idx[:, k:k + 1]
        onehot = cols == jk
        nn = _onehot_gather(onehot.astype(jnp.float32), nfull)
        nnx = nn[:, 0:1]
        nny = nn[:, 1:2]
        nnz = nn[:, 2:3]
        cos = jnp.clip(nnx * zx + nny * zy + nnz * zz, -1.0, 1.0)
        hb = jnp.floor((cos + 1.0) / 2.0 * HIST_BINS).astype(jnp.int32)
        hists.append(jnp.clip(hb, 0, HIST_BINS - 1))
    hist = jnp.concatenate(hists, axis=1)

    lz = lz_ref[0]
    ly = ly_ref[0]
    lx = lx_ref[0]
    r = jnp.sqrt(lx * lx + ly * ly + lz * lz + 1e-12)
    rad_bin = jnp.clip(jnp.floor(r / radeps * RAD_BINS).astype(jnp.int32),
                       0, RAD_BINS - 1)
    cos_el = lz / r
    el_bin = jnp.clip(
        jnp.floor((cos_el + 1.0) / 2.0 * EL_BINS).astype(jnp.int32),
        0, EL_BINS - 1)

    a = ly
    b = lx
    az_pos = jnp.where(b > 0.0,
                       jnp.where(a >= b, 5, 4),
                       jnp.where(a + b > 0.0, 6, 7))
    az_neg = jnp.where(b >= 0.0,
                       jnp.where(a + b >= 0.0, 3, 2),
                       jnp.where(a <= b, 1, 0))
    az_bin = jnp.where(a >= 0.0, az_pos, az_neg).astype(jnp.int32)

    spatial = (az_bin * EL_BINS + el_bin) * RAD_BINS + rad_bin
    bin_idx = spatial * HIST_BINS + hist
    contrib = jnp.maximum(1.0 - r / radeps, 0.0)

    bins = jax.lax.broadcasted_iota(jnp.int32, (1, TOTAL_BINS), 1)
    acc = jnp.zeros((ROWS, TOTAL_BINS), dtype=jnp.float32)
    for k in range(K):
        acc = acc + jnp.where(bins == bin_idx[:, k:k + 1],
                              contrib[:, k:k + 1], 0.0)
    nrm = jnp.sqrt(jnp.sum(acc * acc, axis=1, keepdims=True) + 1e-12)
    out_ref[0] = acc / nrm


@jax.jit
def kernel(vertices):
    B = vertices.shape[0]
    v_t = jnp.transpose(vertices, (0, 2, 1))

    row_blk = lambda c, b: (c, b, 0)
    full_blk = lambda c, b: (c, 0, 0)

    lz, ly, lx, nrm, idx, rmax = pl.pallas_call(
        _lrf_kernel,
        grid=(B, NB),
        in_specs=[
            pl.BlockSpec((1, ROWS, 3), row_blk),
            pl.BlockSpec((1, N, 3), full_blk),
            pl.BlockSpec((1, 3, N), full_blk),
        ],
        out_specs=[
            pl.BlockSpec((1, ROWS, K), row_blk),
            pl.BlockSpec((1, ROWS, K), row_blk),
            pl.BlockSpec((1, ROWS, K), row_blk),
            pl.BlockSpec((1, ROWS, 3), row_blk),
            pl.BlockSpec((1, ROWS, K), row_blk),
            pl.BlockSpec((1, ROWS, 1), row_blk),
        ],
        out_shape=[
            jax.ShapeDtypeStruct((B, N, K), jnp.float32),
            jax.ShapeDtypeStruct((B, N, K), jnp.float32),
            jax.ShapeDtypeStruct((B, N, K), jnp.float32),
            jax.ShapeDtypeStruct((B, N, 3), jnp.float32),
            jax.ShapeDtypeStruct((B, N, K), jnp.int32),
            jax.ShapeDtypeStruct((B, N, 1), jnp.float32),
        ],
    )(vertices, vertices, v_t)

    rmax_r = jnp.transpose(rmax, (0, 2, 1))

    descr = pl.pallas_call(
        _descr_kernel,
        grid=(B, NB),
        in_specs=[
            pl.BlockSpec((1, ROWS, 3), row_blk),
            pl.BlockSpec((1, N, 3), full_blk),
            pl.BlockSpec((1, ROWS, K), row_blk),
            pl.BlockSpec((1, ROWS, K), row_blk),
            pl.BlockSpec((1, ROWS, K), row_blk),
            pl.BlockSpec((1, ROWS, K), row_blk),
            pl.BlockSpec((1, 1, N), full_blk),
        ],
        out_specs=pl.BlockSpec((1, ROWS, TOTAL_BINS), row_blk),
        out_shape=jax.ShapeDtypeStruct((B, N, TOTAL_BINS), jnp.float32),
    )(nrm, nrm, idx, lz, ly, lx, rmax_r)

    return descr


# speedup vs baseline: 2.8908x; 2.8908x over previous
import functools

import jax
import jax.numpy as jnp
from jax.experimental import pallas as pl

N = 2048
K = 16
AZ_BINS = 8
EL_BINS = 2
RAD_BINS = 2
HIST_BINS = 11
TOTAL_BINS = AZ_BINS * EL_BINS * RAD_BINS * HIST_BINS
ROWS = 256
NB = N // ROWS
BIG = 3.0e38


def _eigh3x3(a00, a01, a02, a11, a12, a22, sweeps=5):
    w = {(0, 0): a00, (0, 1): a01, (0, 2): a02,
         (1, 1): a11, (1, 2): a12, (2, 2): a22}
    one = jnp.ones_like(a00)
    zero = jnp.zeros_like(a00)
    v = {}
    for i in range(3):
        for j in range(3):
            v[(i, j)] = one if i == j else zero

    def W(i, j):
        return w[(i, j)] if i <= j else w[(j, i)]

    for _ in range(sweeps):
        for (p, q) in ((0, 2), (2, 1), (0, 1)):
            wpp, wqq, wpq = W(p, p), W(q, q), W(p, q)
            tau = (wqq - wpp) / (2.0 * wpq)
            t = jnp.sign(tau) / (jnp.abs(tau) + jnp.sqrt(1.0 + tau * tau))
            t = jnp.where(wpq != 0.0, t, 0.0)
            c = 1.0 / jnp.sqrt(1.0 + t * t)
            s = t * c
            r = 3 - p - q
            wpr, wqr = W(p, r), W(q, r)
            b_pp = c * wpp - s * wpq
            b_pq = c * wpq - s * wqq
            b_qp = s * wpp + c * wpq
            b_qq = s * wpq + c * wqq
            nw = dict(w)
            nw[(p, p)] = c * b_pp - s * b_pq
            nw[(q, q)] = s * b_qp + c * b_qq
            nw[(min(p, q), max(p, q))] = (s * b_pp + c * b_pq) if p < q \
                else (c * b_qp - s * b_qq)
            nw[(min(p, r), max(p, r))] = c * wpr - s * wqr
            nw[(min(q, r), max(q, r))] = s * wpr + c * wqr
            w = {k: nw[k] for k in w}
            nv = dict(v)
            for i in range(3):
                nv[(i, p)] = c * v[(i, p)] - s * v[(i, q)]
                nv[(i, q)] = s * v[(i, p)] + c * v[(i, q)]
            v = nv

    e = [W(0, 0), W(1, 1), W(2, 2)]
    emin = jnp.minimum(jnp.minimum(e[0], e[1]), e[2])
    emax = jnp.maximum(jnp.maximum(e[0], e[1]), e[2])
    zcol = [jnp.where(e[0] == emin, v[(i, 0)],
                      jnp.where(e[1] == emin, v[(i, 1)], v[(i, 2)]))
            for i in range(3)]
    xcol = [jnp.where(e[2] == emax, v[(i, 2)],
                      jnp.where(e[1] == emax, v[(i, 1)], v[(i, 0)]))
            for i in range(3)]
    return zcol, xcol


def _gather_row(row, idx):
    R = idx.shape[0]
    idx_div = idx >> 7
    idx_mod = idx & 127
    out = jnp.zeros(idx.shape, jnp.float32)
    for g in range(N // 128):
        tab = jnp.broadcast_to(row[:, g * 128:(g + 1) * 128], (R, 128))
        part = jnp.take_along_axis(tab, idx_mod, axis=1)
        out = jnp.where(idx_div == g, part, out)
    return out


def _lrf_kernel(vrow_ref, vt_ref, lz_ref, ly_ref, lx_ref, nrm_ref,
                idx_ref, rmax_ref):
    vb = vrow_ref[0]
    xb = vb[:, 0:1]
    yb = vb[:, 1:2]
    zb = vb[:, 2:3]
    xa = vt_ref[0, 0:1, :]
    ya = vt_ref[0, 1:2, :]
    za = vt_ref[0, 2:3, :]

    d2 = (xb - xa) ** 2 + (yb - ya) ** 2 + (zb - za) ** 2

    row0 = pl.program_id(1) * ROWS
    rows = jax.lax.broadcasted_iota(jnp.int32, (ROWS, 1), 0) + row0
    cols = jax.lax.broadcasted_iota(jnp.int32, (1, N), 1)
    vals = jnp.where(cols == rows, BIG, d2)

    idxs = []
    for _ in range(K):
        m = jnp.min(vals, axis=1, keepdims=True)
        cand = jnp.where(vals == m, cols, N * 2)
        j = jnp.min(cand, axis=1, keepdims=True)
        onehot = cols == j
        vals = jnp.where(onehot, BIG, vals)
        idxs.append(j)
    idx = jnp.concatenate(idxs, axis=1)

    dx = _gather_row(xa, idx) - xb
    dy = _gather_row(ya, idx) - yb
    dz = _gather_row(za, idx) - zb

    dist = jnp.sqrt(dx * dx + dy * dy + dz * dz + 1e-12)
    rloc = jnp.max(dist, axis=1, keepdims=True)
    w = jnp.maximum(rloc - dist, 0.0)
    sw = jnp.sum(w, axis=1, keepdims=True) + 1e-12

    def rne(t):
        return t.astype(jnp.bfloat16).astype(jnp.float32)

    wdx = rne(w * dx)
    wdy = rne(w * dy)
    wdz = rne(w * dz)
    rdx = rne(dx)
    rdy = rne(dy)
    rdz = rne(dz)

    def ksum(t):
        return jnp.sum(t, axis=1, keepdims=True)

    cxx = ksum(wdx * rdx) / sw + 1e-8
    cyy = ksum(wdy * rdy) / sw + 1e-8
    czz = ksum(wdz * rdz) / sw + 1e-8
    cxy = (ksum(wdx * rdy) / sw + ksum(wdy * rdx) / sw) / 2.0
    cxz = (ksum(wdx * rdz) / sw + ksum(wdz * rdx) / sw) / 2.0
    cyz = (ksum(wdy * rdz) / sw + ksum(wdz * rdy) / sw) / 2.0

    zc, xc = _eigh3x3(cxx, cxy, cxz, cyy, cyz, czz)

    dotx = rdx * rne(xc[0]) + rdy * rne(xc[1]) + rdz * rne(xc[2])
    sx = jnp.sum((dotx >= 0.0).astype(jnp.int32), axis=1, keepdims=True)
    fx = jnp.where(2 * sx >= K, 1.0, -1.0)
    xc = [x * fx for x in xc]
    dotz = rdx * rne(zc[0]) + rdy * rne(zc[1]) + rdz * rne(zc[2])
    sz = jnp.sum((dotz >= 0.0).astype(jnp.int32), axis=1, keepdims=True)
    fz = jnp.where(2 * sz >= K, 1.0, -1.0)
    zc = [z * fz for z in zc]

    yc = [zc[1] * xc[2] - zc[2] * xc[1],
          zc[2] * xc[0] - zc[0] * xc[2],
          zc[0] * xc[1] - zc[1] * xc[0]]

    lz = rdx * rne(zc[0]) + rdy * rne(zc[1]) + rdz * rne(zc[2])
    ly = rdx * rne(yc[0]) + rdy * rne(yc[1]) + rdz * rne(yc[2])
    lx = rdx * rne(xc[0]) + rdy * rne(xc[1]) + rdz * rne(xc[2])

    lz_ref[0] = lz
    ly_ref[0] = ly
    lx_ref[0] = lx
    nrm_ref[0] = jnp.concatenate(zc, axis=1)
    idx_ref[0] = idx
    rmax_ref[0] = jnp.max(jnp.maximum(jnp.maximum(lz, ly), lx), axis=1,
                          keepdims=True)


def _descr_kernel(nrow_ref, nt_ref, idx_ref, lz_ref, ly_ref, lx_ref,
                  rmax_ref, out_ref):
    radius = jnp.max(rmax_ref[0], axis=1, keepdims=True)
    radeps = radius + 1e-12

    nb = nrow_ref[0]
    zx = nb[:, 0:1]
    zy = nb[:, 1:2]
    zz = nb[:, 2:3]
    idx = idx_ref[0]

    nnx = _gather_row(nt_ref[0, 0:1, :], idx)
    nny = _gather_row(nt_ref[0, 1:2, :], idx)
    nnz = _gather_row(nt_ref[0, 2:3, :], idx)
    cos = jnp.clip(nnx * zx + nny * zy + nnz * zz, -1.0, 1.0)
    hb = jnp.floor((cos + 1.0) / 2.0 * HIST_BINS).astype(jnp.int32)
    hist = jnp.clip(hb, 0, HIST_BINS - 1)

    lz = lz_ref[0]
    ly = ly_ref[0]
    lx = lx_ref[0]
    r = jnp.sqrt(lx * lx + ly * ly + lz * lz + 1e-12)
    rad_bin = jnp.clip(jnp.floor(r / radeps * RAD_BINS).astype(jnp.int32),
                       0, RAD_BINS - 1)
    cos_el = lz / r
    el_bin = jnp.clip(
        jnp.floor((cos_el + 1.0) / 2.0 * EL_BINS).astype(jnp.int32),
        0, EL_BINS - 1)

    a = ly
    b = lx
    az_pos = jnp.where(b > 0.0,
                       jnp.where(a >= b, 5, 4),
                       jnp.where(a + b > 0.0, 6, 7))
    az_neg = jnp.where(b >= 0.0,
                       jnp.where(a + b >= 0.0, 3, 2),
                       jnp.where(a <= b, 1, 0))
    az_bin = jnp.where(a >= 0.0, az_pos, az_neg).astype(jnp.int32)

    spatial = (az_bin * EL_BINS + el_bin) * RAD_BINS + rad_bin
    bin_idx = spatial * HIST_BINS + hist
    contrib = jnp.maximum(1.0 - r / radeps, 0.0)

    bins = jax.lax.broadcasted_iota(jnp.int32, (1, TOTAL_BINS), 1)
    acc = jnp.zeros((ROWS, TOTAL_BINS), dtype=jnp.float32)
    for k in range(K):
        acc = acc + jnp.where(bins == bin_idx[:, k:k + 1],
                              contrib[:, k:k + 1], 0.0)
    nrm = jnp.sqrt(jnp.sum(acc * acc, axis=1, keepdims=True) + 1e-12)
    out_ref[0] = acc / nrm


@jax.jit
def kernel(vertices):
    B = vertices.shape[0]
    v_t = jnp.transpose(vertices, (0, 2, 1))

    row_blk = lambda c, b: (c, b, 0)
    full_blk = lambda c, b: (c, 0, 0)

    lz, ly, lx, nrm, idx, rmax = pl.pallas_call(
        _lrf_kernel,
        grid=(B, NB),
        in_specs=[
            pl.BlockSpec((1, ROWS, 3), row_blk),
            pl.BlockSpec((1, 3, N), full_blk),
        ],
        out_specs=[
            pl.BlockSpec((1, ROWS, K), row_blk),
            pl.BlockSpec((1, ROWS, K), row_blk),
            pl.BlockSpec((1, ROWS, K), row_blk),
            pl.BlockSpec((1, ROWS, 3), row_blk),
            pl.BlockSpec((1, ROWS, K), row_blk),
            pl.BlockSpec((1, ROWS, 1), row_blk),
        ],
        out_shape=[
            jax.ShapeDtypeStruct((B, N, K), jnp.float32),
            jax.ShapeDtypeStruct((B, N, K), jnp.float32),
            jax.ShapeDtypeStruct((B, N, K), jnp.float32),
            jax.ShapeDtypeStruct((B, N, 3), jnp.float32),
            jax.ShapeDtypeStruct((B, N, K), jnp.int32),
            jax.ShapeDtypeStruct((B, N, 1), jnp.float32),
        ],
    )(vertices, v_t)

    nrm_t = jnp.transpose(nrm, (0, 2, 1))
    rmax_r = jnp.transpose(rmax, (0, 2, 1))

    descr = pl.pallas_call(
        _descr_kernel,
        grid=(B, NB),
        in_specs=[
            pl.BlockSpec((1, ROWS, 3), row_blk),
            pl.BlockSpec((1, 3, N), full_blk),
            pl.BlockSpec((1, ROWS, K), row_blk),
            pl.BlockSpec((1, ROWS, K), row_blk),
            pl.BlockSpec((1, ROWS, K), row_blk),
            pl.BlockSpec((1, ROWS, K), row_blk),
            pl.BlockSpec((1, 1, N), full_blk),
        ],
        out_specs=pl.BlockSpec((1, ROWS, TOTAL_BINS), row_blk),
        out_shape=jax.ShapeDtypeStruct((B, N, TOTAL_BINS), jnp.float32),
    )(nrm, nrm_t, idx, lz, ly, lx, rmax_r)

    return descr
